# baseline (speedup 1.0000x reference)
import functools

import jax
import jax.numpy as jnp
from jax import lax
from jax.experimental import pallas as pl
from jax.experimental.pallas import tpu as pltpu
from jax.experimental.pallas import tpu_sc as plsc

B = 16384
F = 64
NC = 2
NS = 16
NW = NC * NS
BPW = B // NW
CHUNK = 128
NCHUNK = BPW // CHUNK
BCH = 64
NBCH = BPW // BCH

N_U = 100000
N_I = 1000000


def _pad_rows(n):
    return (n + 127) // 128


def _mf_body(u_idx_hbm, i_idx_hbm, u_emb_hbm, i_emb_hbm, u_bias_hbm,
             i_bias_hbm, out_hbm,
             uidx_v, iidx_v, urow_v, irow_v, u_rows, i_rows,
             ubr0, ibr0, ubr1, ibr1, ubval, ibval, out_v,
             sem_e, sb0, sb1):
    cid = lax.axis_index("c")
    sid = lax.axis_index("s")
    wid = sid * NC + cid
    base = wid * BPW

    pltpu.sync_copy(u_idx_hbm.at[pl.ds(base, BPW)], uidx_v)
    pltpu.sync_copy(i_idx_hbm.at[pl.ds(base, BPW)], iidx_v)

    def shift_body(g, carry):
        sl = pl.ds(g * 16, 16)
        urow_v[sl] = uidx_v[sl] >> 7
        irow_v[sl] = iidx_v[sl] >> 7
        return carry

    lax.fori_loop(0, BPW // 16, shift_body, 0)

    emb_copies = []
    for c in range(NCHUNK):
        sl = pl.ds(c * CHUNK, CHUNK)
        emb_copies.append(pltpu.async_copy(
            u_emb_hbm.at[uidx_v.at[sl]], u_rows.at[sl], sem_e))
        emb_copies.append(pltpu.async_copy(
            i_emb_hbm.at[iidx_v.at[sl]], i_rows.at[sl], sem_e))

    lane = lax.iota(jnp.int32, 16)
    ring = ((ubr0, ibr0, sb0), (ubr1, ibr1, sb1))

    def fire_bias(c):
        ubr, ibr, sb = ring[c % 2]
        sl = pl.ds(c * BCH, BCH)
        return (pltpu.async_copy(u_bias_hbm.at[urow_v.at[sl]], ubr, sb),
                pltpu.async_copy(i_bias_hbm.at[irow_v.at[sl]], ibr, sb))

    pending = {0: fire_bias(0), 1: fire_bias(1)}
    for c in range(NBCH):
        ubr, ibr, _ = ring[c % 2]
        for cp in pending.pop(c):
            cp.wait()
        for g in range(BCH // 16):
            sl = pl.ds(c * BCH + g * 16, 16)
            bvec = g * 16 + lane
            ubval[sl] = plsc.load_gather(ubr, [bvec, uidx_v[sl] & 127])
            ibval[sl] = plsc.load_gather(ibr, [bvec, iidx_v[sl] & 127])
        if c + 2 < NBCH:
            pending[c + 2] = fire_bias(c + 2)

    for cp in emb_copies:
        cp.wait()

    def body(g, carry):
        sl = pl.ds(g * 16, 16)
        res = ubval[sl] + ibval[sl]
        for j in range(16):
            b = g * 16 + j
            acc = u_rows[b, pl.ds(0, 16)] * i_rows[b, pl.ds(0, 16)]
            for c in range(1, F // 16):
                acc = acc + u_rows[b, pl.ds(c * 16, 16)] * i_rows[b, pl.ds(c * 16, 16)]
            res = res + jnp.where(lane == j, jnp.sum(acc), 0.0)
        out_v[sl] = res
        return carry

    lax.fori_loop(0, BPW // 16, body, 0)

    pltpu.sync_copy(out_v, out_hbm.at[pl.ds(base, BPW)])


def _mf(u_idx, i_idx, u_emb, i_emb, u_bias2, i_bias2):
    mesh = plsc.VectorSubcoreMesh(core_axis_name="c", subcore_axis_name="s")
    f = functools.partial(
        pl.kernel,
        out_type=jax.ShapeDtypeStruct((B,), jnp.float32),
        mesh=mesh,
        scratch_types=[
            pltpu.VMEM((BPW,), jnp.int32),
            pltpu.VMEM((BPW,), jnp.int32),
            pltpu.VMEM((BPW,), jnp.int32),
            pltpu.VMEM((BPW,), jnp.int32),
            pltpu.VMEM((BPW, F), jnp.float32),
            pltpu.VMEM((BPW, F), jnp.float32),
            pltpu.VMEM((BCH, 128), jnp.float32),
            pltpu.VMEM((BCH, 128), jnp.float32),
            pltpu.VMEM((BCH, 128), jnp.float32),
            pltpu.VMEM((BCH, 128), jnp.float32),
            pltpu.VMEM((BPW,), jnp.float32),
            pltpu.VMEM((BPW,), jnp.float32),
            pltpu.VMEM((BPW,), jnp.float32),
            pltpu.SemaphoreType.DMA,
            pltpu.SemaphoreType.DMA,
            pltpu.SemaphoreType.DMA,
        ],
        compiler_params=pltpu.CompilerParams(
            needs_layout_passes=False, use_tc_tiling_on_sc=False),
    )(_mf_body)
    return f(u_idx, i_idx, u_emb, i_emb, u_bias2, i_bias2)


def _bias_rows(bias, n):
    rows = _pad_rows(n)
    flat = jnp.squeeze(bias, -1)
    flat = jnp.pad(flat, (0, rows * 128 - n))
    return flat.reshape(rows, 128)


def kernel(u_idx, i_idx, u_emb, i_emb, u_bias, i_bias):
    u_bias2 = _bias_rows(u_bias, N_U)
    i_bias2 = _bias_rows(i_bias, N_I)
    return _mf(u_idx.astype(jnp.int32), i_idx.astype(jnp.int32),
               u_emb, i_emb, u_bias2, i_bias2)


# speedup vs baseline: 1.0133x; 1.0133x over previous
import functools

import jax
import jax.numpy as jnp
from jax import lax
from jax.experimental import pallas as pl
from jax.experimental.pallas import tpu as pltpu
from jax.experimental.pallas import tpu_sc as plsc

B = 16384
F = 64
NC = 2
NS = 16
NW = NC * NS
BPW = B // NW
CHUNK = 128
NCHUNK = BPW // CHUNK


def _mf_body(u_idx_hbm, i_idx_hbm, u_emb_hbm, i_emb_hbm, u_bias_hbm,
             i_bias_hbm, out_hbm,
             uidx_v, iidx_v, u_rows, i_rows, ubval, ibval, out_v, sem):
    cid = lax.axis_index("c")
    sid = lax.axis_index("s")
    wid = sid * NC + cid
    base = wid * BPW

    pltpu.sync_copy(u_idx_hbm.at[pl.ds(base, BPW)], uidx_v)
    pltpu.sync_copy(i_idx_hbm.at[pl.ds(base, BPW)], iidx_v)

    copies = []
    for c in range(NCHUNK):
        sl = pl.ds(c * CHUNK, CHUNK)
        copies.append(pltpu.async_copy(
            u_emb_hbm.at[uidx_v.at[sl]], u_rows.at[sl], sem))
        copies.append(pltpu.async_copy(
            i_emb_hbm.at[iidx_v.at[sl]], i_rows.at[sl], sem))
        copies.append(pltpu.async_copy(
            u_bias_hbm.at[uidx_v.at[sl]], ubval.at[sl], sem))
        copies.append(pltpu.async_copy(
            i_bias_hbm.at[iidx_v.at[sl]], ibval.at[sl], sem))
    for cp in copies:
        cp.wait()

    lane = lax.iota(jnp.int32, 16)

    def body(g, carry):
        sl = pl.ds(g * 16, 16)
        res = ubval[sl] + ibval[sl]
        for j in range(16):
            b = g * 16 + j
            acc = u_rows[b, pl.ds(0, 16)] * i_rows[b, pl.ds(0, 16)]
            for c in range(1, F // 16):
                acc = acc + u_rows[b, pl.ds(c * 16, 16)] * i_rows[b, pl.ds(c * 16, 16)]
            res = res + jnp.where(lane == j, jnp.sum(acc), 0.0)
        out_v[sl] = res
        return carry

    lax.fori_loop(0, BPW // 16, body, 0)

    pltpu.sync_copy(out_v, out_hbm.at[pl.ds(base, BPW)])


def _mf(u_idx, i_idx, u_emb, i_emb, u_bias1, i_bias1):
    mesh = plsc.VectorSubcoreMesh(core_axis_name="c", subcore_axis_name="s")
    f = functools.partial(
        pl.kernel,
        out_type=jax.ShapeDtypeStruct((B,), jnp.float32),
        mesh=mesh,
        scratch_types=[
            pltpu.VMEM((BPW,), jnp.int32),
            pltpu.VMEM((BPW,), jnp.int32),
            pltpu.VMEM((BPW, F), jnp.float32),
            pltpu.VMEM((BPW, F), jnp.float32),
            pltpu.VMEM((BPW,), jnp.float32),
            pltpu.VMEM((BPW,), jnp.float32),
            pltpu.VMEM((BPW,), jnp.float32),
            pltpu.SemaphoreType.DMA,
        ],
        compiler_params=pltpu.CompilerParams(
            needs_layout_passes=False, use_tc_tiling_on_sc=False),
    )(_mf_body)
    return f(u_idx, i_idx, u_emb, i_emb, u_bias1, i_bias1)


def kernel(u_idx, i_idx, u_emb, i_emb, u_bias, i_bias):
    return _mf(u_idx.astype(jnp.int32), i_idx.astype(jnp.int32),
               u_emb, i_emb,
               jnp.squeeze(u_bias, -1), jnp.squeeze(i_bias, -1))


# speedup vs baseline: 1.0140x; 1.0007x over previous
import functools

import jax
import jax.numpy as jnp
from jax import lax
from jax.experimental import pallas as pl
from jax.experimental.pallas import tpu as pltpu
from jax.experimental.pallas import tpu_sc as plsc

B = 16384
F = 64
NC = 2
NS = 16
NW = NC * NS
BPW = B // NW
CHUNK = 128
NCHUNK = BPW // CHUNK


def _mf_body(u_idx_hbm, i_idx_hbm, u_emb_hbm, i_emb_hbm, u_bias_hbm,
             i_bias_hbm, out_hbm,
             uidx_v, iidx_v, u_rows, i_rows, ubval, ibval, out_v, sem):
    cid = lax.axis_index("c")
    sid = lax.axis_index("s")
    wid = sid * NC + cid
    base = wid * BPW

    pltpu.sync_copy(u_idx_hbm.at[pl.ds(base, BPW)], uidx_v)
    pltpu.sync_copy(i_idx_hbm.at[pl.ds(base, BPW)], iidx_v)

    copies = []
    for c in range(NCHUNK):
        sl = pl.ds(c * CHUNK, CHUNK)
        copies.append(pltpu.async_copy(
            u_emb_hbm.at[uidx_v.at[sl]], u_rows.at[sl], sem))
        copies.append(pltpu.async_copy(
            i_emb_hbm.at[iidx_v.at[sl]], i_rows.at[sl], sem))
        copies.append(pltpu.async_copy(
            u_bias_hbm.at[uidx_v.at[sl]], ubval.at[sl], sem))
        copies.append(pltpu.async_copy(
            i_bias_hbm.at[iidx_v.at[sl]], ibval.at[sl], sem))
    for cp in copies:
        cp.wait()

    lane = lax.iota(jnp.int32, 16)

    def body(g, carry):
        sl = pl.ds(g * 16, 16)
        res = ubval[sl] + ibval[sl]
        for j in range(16):
            b = g * 16 + j
            acc = u_rows[b, pl.ds(0, 16)] * i_rows[b, pl.ds(0, 16)]
            for c in range(1, F // 16):
                acc = acc + u_rows[b, pl.ds(c * 16, 16)] * i_rows[b, pl.ds(c * 16, 16)]
            res = res + jnp.where(lane == j, jnp.sum(acc), 0.0)
        out_v[sl] = res
        return carry

    lax.fori_loop(0, BPW // 16, body, 0)

    pltpu.sync_copy(out_v, out_hbm.at[pl.ds(base, BPW)])


def _mf(u_idx, i_idx, u_emb, i_emb, u_bias1, i_bias1):
    mesh = plsc.VectorSubcoreMesh(core_axis_name="c", subcore_axis_name="s")
    f = functools.partial(
        pl.kernel,
        out_type=jax.ShapeDtypeStruct((B,), jnp.float32),
        mesh=mesh,
        scratch_types=[
            pltpu.VMEM((BPW,), jnp.int32),
            pltpu.VMEM((BPW,), jnp.int32),
            pltpu.VMEM((BPW, F), jnp.float32),
            pltpu.VMEM((BPW, F), jnp.float32),
            pltpu.VMEM((BPW,), jnp.float32),
            pltpu.VMEM((BPW,), jnp.float32),
            pltpu.VMEM((BPW,), jnp.float32),
            pltpu.SemaphoreType.DMA,
        ],
        compiler_params=pltpu.CompilerParams(
            needs_layout_passes=False, use_tc_tiling_on_sc=False),
    )(_mf_body)
    return f(u_idx, i_idx, u_emb, i_emb, u_bias1, i_bias1)


def kernel(u_idx, i_idx, u_emb, i_emb, u_bias, i_bias):
    u_bias1 = lax.reshape(u_bias, (u_bias.shape[0],), dimensions=(1, 0))
    i_bias1 = lax.reshape(i_bias, (i_bias.shape[0],), dimensions=(1, 0))
    return _mf(u_idx.astype(jnp.int32), i_idx.astype(jnp.int32),
               u_emb, i_emb, u_bias1, i_bias1)
